# baseline (speedup 1.0000x reference)
import functools

import jax
import jax.numpy as jnp
import numpy as np
from jax import lax
from jax.experimental import pallas as pl
from jax.experimental.pallas import tpu as pltpu
from jax.experimental.pallas import tpu_sc as plsc

VOCAB = 1000000
D = 64
BATCH = 4096
SEQ = 200

NC = 2
NS = 16
NW = NC * NS
LANES = 16

BT = BATCH // NW
DT = D // 8
LB = BT // LANES
DJ = D // LANES


def _compute_encoding(max_len, d):
    enc = np.zeros((max_len, d), dtype=np.float32)
    pos = np.arange(0, max_len, dtype=np.float32)
    for i in range(d // 2):
        enc[:, 2 * i] = np.sin(pos / 10000 ** (2 * i / d))
        enc[:, 2 * i + 1] = np.cos(pos / 10000 ** (2 * i / d))
    return enc


_ENC = _compute_encoding(SEQ, D)


def _lane_shuffle(vec, idx):
    return lax.gather(
        vec, idx[:, None],
        lax.GatherDimensionNumbers(
            offset_dims=(), collapsed_slice_dims=(0,), start_index_map=(0,)),
        slice_sizes=(1,),
        mode=lax.GatherScatterMode.PROMISE_IN_BOUNDS)


def _body(xt_hbm, tab_hbm, enc_hbm, out_hbm,
          idx_v, emb_v, out_v, enc_v, idx_sem, gat_sem, out_sem):
    w = lax.axis_index("s") * NC + lax.axis_index("c")

    pltpu.sync_copy(enc_hbm, enc_v)

    def idx_start(s, b):
        pltpu.make_async_copy(
            xt_hbm.at[s, pl.ds(w * BT, BT)], idx_v.at[b], idx_sem.at[b]).start()

    def idx_wait(b):
        pltpu.make_async_copy(
            xt_hbm.at[0, pl.ds(0, BT)], idx_v.at[b], idx_sem.at[b]).wait()

    def gather_start(b):
        pltpu.make_async_copy(
            tab_hbm.at[idx_v.at[b]], emb_v.at[b], gat_sem.at[b]).start()

    def gather_wait(b):
        pltpu.make_async_copy(
            tab_hbm.at[pl.ds(0, BT)], emb_v.at[b], gat_sem.at[b]).wait()

    def out_start(s, b):
        pltpu.make_async_copy(
            out_v.at[b], out_hbm.at[s, :, w], out_sem.at[b]).start()

    def out_wait(b):
        pltpu.make_async_copy(
            out_v.at[b], out_hbm.at[0, :, 0], out_sem.at[b]).wait()

    iota = lax.iota(jnp.int32, LANES)
    rows = [lb * LANES + iota for lb in range(LB)]
    three = jnp.int32(3)
    seven = jnp.int32(7)
    fifteen = jnp.int32(LANES - 1)

    def compute(s, b):
        evecs = [enc_v[s, pl.ds(LANES * dj, LANES)] for dj in range(DJ)]

        def k_body(k, _):
            rot = lax.bitwise_and(iota + k, fifteen)
            for dj in range(DJ):
                dval = rot + (LANES * dj)
                dtv = lax.shift_right_logical(dval, three)
                d8v = lax.bitwise_and(dval, seven)
                e_rot = _lane_shuffle(evecs[dj], rot)
                for lb in range(LB):
                    g = plsc.load_gather(emb_v.at[b], [rows[lb], dval])
                    plsc.store_scatter(
                        out_v.at[b], [dtv, d8v, rows[lb]], g + e_rot)
            return ()

        lax.fori_loop(0, LANES, k_body, (), unroll=8)

    idx_start(0, 0)
    idx_wait(0)
    gather_start(0)
    idx_start(1, 1)

    def pair_body(s2, _):
        for b in (0, 1):
            s = 2 * s2 + b
            nb = 1 - b

            @pl.when(s >= 2)
            def _():
                out_wait(b)

            gather_wait(b)

            @pl.when(s + 1 < SEQ)
            def _():
                idx_wait(nb)
                gather_start(nb)

            @pl.when(s + 2 < SEQ)
            def _():
                idx_start(s + 2, b)

            compute(s, b)
            out_start(s, b)
        return ()

    lax.fori_loop(0, SEQ // 2, pair_body, (), unroll=False)

    out_wait(0)
    out_wait(1)


@jax.jit
def kernel(x, table):
    xt = x.T
    enc = jnp.asarray(_ENC)
    mesh = plsc.VectorSubcoreMesh(core_axis_name="c", subcore_axis_name="s")
    out5d = pl.kernel(
        _body,
        out_type=jax.ShapeDtypeStruct((SEQ, DT, NW, 8, BT), jnp.float32),
        mesh=mesh,
        compiler_params=pltpu.CompilerParams(
            use_tc_tiling_on_sc=False, needs_layout_passes=False),
        scratch_types=[
            pltpu.VMEM((2, BT), jnp.int32),
            pltpu.VMEM((2, BT, D), jnp.float32),
            pltpu.VMEM((2, DT, 8, BT), jnp.float32),
            pltpu.VMEM((SEQ, D), jnp.float32),
            pltpu.SemaphoreType.DMA((2,)),
            pltpu.SemaphoreType.DMA((2,)),
            pltpu.SemaphoreType.DMA((2,)),
        ],
    )(xt, table, enc)
    return out5d.transpose(2, 4, 0, 1, 3).reshape(BATCH, SEQ, D)


# speedup vs baseline: 1.4899x; 1.4899x over previous
import functools

import jax
import jax.numpy as jnp
import numpy as np
from jax import lax
from jax.experimental import pallas as pl
from jax.experimental.pallas import tpu as pltpu
from jax.experimental.pallas import tpu_sc as plsc

VOCAB = 1000000
D = 64
BATCH = 4096
SEQ = 200

NC = 2
NS = 16
NW = NC * NS
LANES = 16

BT = BATCH // NW
DT = D // 8
LB = BT // LANES
DJ = D // LANES


def _compute_encoding(max_len, d):
    enc = np.zeros((max_len, d), dtype=np.float32)
    pos = np.arange(0, max_len, dtype=np.float32)
    for i in range(d // 2):
        enc[:, 2 * i] = np.sin(pos / 10000 ** (2 * i / d))
        enc[:, 2 * i + 1] = np.cos(pos / 10000 ** (2 * i / d))
    return enc


_ENC = _compute_encoding(SEQ, D)


def _lane_shuffle(vec, idx):
    return lax.gather(
        vec, idx[:, None],
        lax.GatherDimensionNumbers(
            offset_dims=(), collapsed_slice_dims=(0,), start_index_map=(0,)),
        slice_sizes=(1,),
        mode=lax.GatherScatterMode.PROMISE_IN_BOUNDS)


def _body(xt_hbm, tab_hbm, enc_hbm, out_hbm,
          idx_v, emb_v, out_v, enc_v, idx_sem, gat_sem, out_sem):
    w = lax.axis_index("s") * NC + lax.axis_index("c")

    pltpu.sync_copy(enc_hbm, enc_v)

    def idx_start(s, b):
        pltpu.make_async_copy(
            xt_hbm.at[s, pl.ds(w * BT, BT)], idx_v.at[b], idx_sem.at[b]).start()

    def idx_wait(b):
        pltpu.make_async_copy(
            xt_hbm.at[0, pl.ds(0, BT)], idx_v.at[b], idx_sem.at[b]).wait()

    def gather_start(b):
        pltpu.make_async_copy(
            tab_hbm.at[idx_v.at[b]], emb_v.at[b], gat_sem.at[b]).start()

    def gather_wait(b):
        pltpu.make_async_copy(
            tab_hbm.at[pl.ds(0, BT)], emb_v.at[b], gat_sem.at[b]).wait()

    def out_start(s, b):
        pltpu.make_async_copy(
            out_v.at[b], out_hbm.at[s, :, w], out_sem.at[b]).start()

    def out_wait(b):
        pltpu.make_async_copy(
            out_v.at[b], out_hbm.at[0, :, 0], out_sem.at[b]).wait()

    iota = lax.iota(jnp.int32, LANES)
    rows = [lb * LANES + iota for lb in range(LB)]
    three = jnp.int32(3)
    seven = jnp.int32(7)
    fifteen = jnp.int32(LANES - 1)

    def compute(s, b):
        evecs = [enc_v[s, pl.ds(LANES * dj, LANES)] for dj in range(DJ)]

        def k_body(k, _):
            rot = lax.bitwise_and(iota + k, fifteen)
            for dj in range(DJ):
                dval = rot + (LANES * dj)
                dtv = lax.shift_right_logical(dval, three)
                d8v = lax.bitwise_and(dval, seven)
                e_rot = _lane_shuffle(evecs[dj], rot)
                for lb in range(LB):
                    g = plsc.load_gather(emb_v.at[b], [rows[lb], dval])
                    plsc.store_scatter(
                        out_v.at[b], [dtv, d8v, rows[lb]], g + e_rot)
            return ()

        lax.fori_loop(0, LANES, k_body, (), unroll=4)

    idx_start(0, 0)
    idx_wait(0)
    gather_start(0)
    idx_start(1, 1)

    def pair_body(s2, _):
        for b in (0, 1):
            s = 2 * s2 + b
            nb = 1 - b

            @pl.when(s >= 2)
            def _():
                out_wait(b)

            gather_wait(b)

            @pl.when(s + 1 < SEQ)
            def _():
                idx_wait(nb)
                gather_start(nb)

            @pl.when(s + 2 < SEQ)
            def _():
                idx_start(s + 2, b)

            compute(s, b)
            out_start(s, b)
        return ()

    lax.fori_loop(0, SEQ // 2, pair_body, (), unroll=False)

    out_wait(0)
    out_wait(1)


@jax.jit
def kernel(x, table):
    xt = x.T
    enc = jnp.asarray(_ENC)
    mesh = plsc.VectorSubcoreMesh(core_axis_name="c", subcore_axis_name="s")
    out5d = pl.kernel(
        _body,
        out_type=jax.ShapeDtypeStruct((SEQ, DT, NW, 8, BT), jnp.float32),
        mesh=mesh,
        compiler_params=pltpu.CompilerParams(
            use_tc_tiling_on_sc=False, needs_layout_passes=False),
        scratch_types=[
            pltpu.VMEM((2, BT), jnp.int32),
            pltpu.VMEM((2, BT, D), jnp.float32),
            pltpu.VMEM((2, DT, 8, BT), jnp.float32),
            pltpu.VMEM((SEQ, D), jnp.float32),
            pltpu.SemaphoreType.DMA((2,)),
            pltpu.SemaphoreType.DMA((2,)),
            pltpu.SemaphoreType.DMA((2,)),
        ],
    )(xt, table, enc)
    return out5d.transpose(2, 4, 0, 1, 3).reshape(BATCH, SEQ, D)
